# Initial kernel scaffold; baseline (speedup 1.0000x reference)
import jax
import jax.numpy as jnp
from jax.experimental import pallas as pl


def kernel(global_vec, batch, params):
    raise NotImplementedError("write your pallas kernel here")


# speedup vs baseline: 30.9516x; 30.9516x over previous
import functools

import jax
import jax.numpy as jnp
import numpy as np
from jax.experimental import pallas as pl
from jax.experimental.pallas import tpu as pltpu

NPG = 9
EPS = 1e-5
SLOPE = 0.01


def _lap_eig():
    A = np.ones((NPG, NPG)) - np.eye(NPG)
    L = np.diag(A.sum(axis=1)) - A
    _, v = np.linalg.eig(L)
    v = np.real(v)
    v = v / np.linalg.norm(v, axis=0)
    return jnp.asarray(v, jnp.float32)


def _leaky(x):
    return jnp.where(x >= 0, x, SLOPE * x)


def _full_spec(shape):
    nd = len(shape)
    return pl.BlockSpec(shape, lambda i: (0,) * nd)


def _stats(ps_ref, n_rows, blk_rows, nm):
    ps = ps_ref[...]
    s, sd_, sd2 = ps[:, 0], ps[:, 1], ps[:, 2]
    mean = (blk_rows * jnp.sum(s, axis=0, keepdims=True)
            + jnp.sum(sd_, axis=0, keepdims=True)) / n_rows
    delta = mean * nm - s
    var = (jnp.sum(sd2, axis=0, keepdims=True)
           - 2.0 * jnp.sum(delta * sd_, axis=0, keepdims=True)
           + blk_rows * jnp.sum(delta * delta, axis=0, keepdims=True)) / n_rows
    return mean, jnp.sqrt(var + EPS)


def _shifted_partials(hp):
    s = jnp.mean(hp, axis=0, keepdims=True)
    d = hp - s
    return jnp.stack([s[0], jnp.sum(d, axis=0), jnp.sum(d * d, axis=0)])[None]


def _p0_kernel(gv_ref, nw1, nb1, nw2, nb2, nw3, nb3,
               gw1, gb1, gw2, gb2, gw3, gb3, w1a,
               pred_ref, u_ref, ps_ref):
    x = gv_ref[...]
    a = jnp.maximum(jnp.dot(x, nw1[...], preferred_element_type=jnp.float32) + nb1[...], 0.0)
    a = jnp.maximum(jnp.dot(a, nw2[...], preferred_element_type=jnp.float32) + nb2[...], 0.0)
    pred_ref[...] = jnp.dot(a, nw3[...], preferred_element_type=jnp.float32) + nb3[...]
    g = _leaky(jnp.dot(x, gw1[...], preferred_element_type=jnp.float32) + gb1[...])
    g = _leaky(jnp.dot(g, gw2[...], preferred_element_type=jnp.float32) + gb2[...])
    g = jnp.dot(g, gw3[...], preferred_element_type=jnp.float32) + gb3[...]
    agg = g + g
    for _ in range(7):
        agg = agg + g
    u = jnp.dot(g + agg, w1a[...], preferred_element_type=jnp.float32)
    u_ref[...] = u
    ps_ref[...] = jnp.stack([jnp.sum(u, axis=0), jnp.sum(u * u, axis=0)])[None]


def _gin_tail(t, w2, b2, pw, pb, pm, w1n, b1n, g_blk):
    h = jnp.dot(t.reshape(NPG * g_blk, 128), w2[...],
                preferred_element_type=jnp.float32) + b2[...]
    h3 = h.reshape(NPG, g_blk, 128)
    mg = jnp.mean(h3, axis=0, keepdims=True)
    out = h3 - mg * pm[...][None]
    vg = jnp.mean(out * out, axis=0, keepdims=True)
    y = _leaky(pw[...][None] * out / jnp.sqrt(vg + EPS) + pb[...][None])
    hin = y + jnp.sum(y, axis=0, keepdims=True)
    return jnp.dot(hin.reshape(NPG * g_blk, 128), w1n[...],
                   preferred_element_type=jnp.float32) + b1n[...]


def _p1_kernel(u_ref, c_ref, ps0_ref,
               n0w, n0b, n0m, w2_0, b2_0, pw, pb, pm, w1_1, b1_1,
               hp1_ref, ps_ref, *, batch_size, g_blk):
    nm = n0m[...]
    c = c_ref[...]
    sums = jnp.sum(ps0_ref[...], axis=0)
    su, su2 = sums[0:1], sums[1:2]
    sc = jnp.sum(c, axis=0, keepdims=True)
    sc2 = jnp.sum(c * c, axis=0, keepdims=True)
    n_rows = batch_size * NPG
    mean = (NPG * su + batch_size * sc) / n_rows
    eh2 = (NPG * su2 + 2.0 * su * sc + batch_size * sc2) / n_rows
    var = eh2 - mean * mean * nm * (2.0 - nm)
    sd = jnp.sqrt(var + EPS)

    hpre0 = u_ref[...][None, :, :] + c[:, None, :]
    out0 = hpre0 - (mean * nm)[None]
    t = _leaky(n0w[...][None] * out0 / sd[None] + n0b[...][None])
    hp1 = _gin_tail(t, w2_0, b2_0, pw, pb, pm, w1_1, b1_1, g_blk)
    hp1_ref[...] = hp1.reshape(NPG, g_blk, 128)
    ps_ref[...] = _shifted_partials(hp1)


def _p2_kernel(hp_ref, ps_in_ref,
               nw, nb, nm_, w2, b2, pw, pb, pm, w1n, b1n,
               hp_out_ref, ps_ref, *, n_rows, g_blk):
    nm = nm_[...]
    mean, sd = _stats(ps_in_ref, n_rows, NPG * g_blk, nm)
    out0 = hp_ref[...] - (mean * nm)[None]
    t = _leaky(nw[...][None] * out0 / sd[None] + nb[...][None])
    hp = _gin_tail(t, w2, b2, pw, pb, pm, w1n, b1n, g_blk)
    hp_out_ref[...] = hp.reshape(NPG, g_blk, 128)
    ps_ref[...] = _shifted_partials(hp)


def _p3_kernel(hp_ref, ps_in_ref,
               nw, nb, nm_, w2, b2,
               fw1, fb1, fw2, fb2, fw3, fb3,
               ew1, eb1, ew2, eb2, ew3, eb3,
               node_ref, edgef_ref, *, n_rows, g_blk, stat_blk_rows):
    nm = nm_[...]
    mean, sd = _stats(ps_in_ref, n_rows, stat_blk_rows, nm)
    out0 = hp_ref[...] - (mean * nm)[None]
    t = _leaky(nw[...][None] * out0 / sd[None] + nb[...][None])
    h = jnp.dot(t.reshape(NPG * g_blk, 128), w2[...],
                preferred_element_type=jnp.float32) + b2[...]
    a = _leaky(jnp.dot(h, fw1[...], preferred_element_type=jnp.float32) + fb1[...])
    a = _leaky(jnp.dot(a, fw2[...], preferred_element_type=jnp.float32) + fb2[...])
    node = jnp.dot(a, fw3[...], preferred_element_type=jnp.float32) + fb3[...]
    node_ref[...] = node.reshape(NPG, g_blk, 4)
    h3 = h.reshape(NPG, g_blk, 128)
    for i in range(NPG):
        pin = ((h3 + h3[i:i + 1]) / 2.0).reshape(NPG * g_blk, 128)
        e = _leaky(jnp.dot(pin, ew1[...], preferred_element_type=jnp.float32) + eb1[...])
        e = _leaky(jnp.dot(e, ew2[...], preferred_element_type=jnp.float32) + eb2[...])
        e5 = jnp.dot(e, ew3[...], preferred_element_type=jnp.float32) + eb3[...]
        edgef_ref[i] = e5.reshape(NPG, g_blk, 5)


def _row2(v):
    return v.reshape(1, -1)


def kernel(global_vec, batch, params):
    del batch
    b = global_vec.shape[0]
    n = b * NPG
    f32 = jnp.float32

    gin0, gin1, gin2 = params['gin']
    (nw1, nb1), (nw2, nb2), (nw3, nb3) = params['num_net']
    (gw1, gb1), (gw2, gb2), (gw3, gb3) = params['glob']
    (p0w, p0b, p0m), (p1w, p1b, p1m) = params['norms']
    (fw1, fb1), (fw2, fb2), (fw3, fb3) = params['feat']
    (ew1, eb1), (ew2, eb2), (ew3, eb3) = params['edge']

    eig = _lap_eig()
    w1a = gin0['w1'][:64]
    c = jnp.dot(eig + jnp.sum(eig, axis=0, keepdims=True),
                gin0['w1'][64:]) + gin0['b1']

    g0 = min(2048, b)
    g1 = min(512, b)
    g3 = min(128, b)
    nblk0 = b // g0
    nblk1 = b // g1
    nblk3 = b // g3

    pred, u, ps0 = pl.pallas_call(
        _p0_kernel,
        grid=(nblk0,),
        in_specs=[pl.BlockSpec((g0, 128), lambda i: (i, 0))] + [
            _full_spec(s) for s in [(128, 128), (1, 128), (128, 128), (1, 128),
                                    (128, 1), (1, 1),
                                    (128, 64), (1, 64), (64, 64), (1, 64),
                                    (64, 64), (1, 64), (64, 128)]],
        out_specs=[pl.BlockSpec((g0, 1), lambda i: (i, 0)),
                   pl.BlockSpec((g0, 128), lambda i: (i, 0)),
                   pl.BlockSpec((1, 2, 128), lambda i: (i, 0, 0))],
        out_shape=[jax.ShapeDtypeStruct((b, 1), f32),
                   jax.ShapeDtypeStruct((b, 128), f32),
                   jax.ShapeDtypeStruct((nblk0, 2, 128), f32)],
        compiler_params=pltpu.CompilerParams(dimension_semantics=("parallel",)),
    )(global_vec, nw1, _row2(nb1), nw2, _row2(nb2), nw3, _row2(nb3),
      gw1, _row2(gb1), gw2, _row2(gb2), gw3, _row2(gb3), w1a)

    hp1, ps1 = pl.pallas_call(
        functools.partial(_p1_kernel, batch_size=b, g_blk=g1),
        grid=(nblk1,),
        in_specs=[pl.BlockSpec((g1, 128), lambda i: (i, 0)),
                  _full_spec((NPG, 128)),
                  _full_spec((nblk0, 2, 128))] + [
            _full_spec(s) for s in [(1, 128)] * 3 + [(128, 128), (1, 128)]
                                    + [(1, 128)] * 3 + [(128, 128), (1, 128)]],
        out_specs=[pl.BlockSpec((NPG, g1, 128), lambda i: (0, i, 0)),
                   pl.BlockSpec((1, 3, 128), lambda i: (i, 0, 0))],
        out_shape=[jax.ShapeDtypeStruct((NPG, b, 128), f32),
                   jax.ShapeDtypeStruct((nblk1, 3, 128), f32)],
        compiler_params=pltpu.CompilerParams(dimension_semantics=("parallel",)),
    )(u, c, ps0,
      _row2(gin0['nw']), _row2(gin0['nb']), _row2(gin0['nm']), gin0['w2'], _row2(gin0['b2']),
      _row2(p0w), _row2(p0b), _row2(p0m), gin1['w1'], _row2(gin1['b1']))

    hp2, ps2 = pl.pallas_call(
        functools.partial(_p2_kernel, n_rows=n, g_blk=g1),
        grid=(nblk1,),
        in_specs=[pl.BlockSpec((NPG, g1, 128), lambda i: (0, i, 0)),
                  _full_spec((nblk1, 3, 128))] + [
            _full_spec(s) for s in [(1, 128)] * 3 + [(128, 128), (1, 128)]
                                    + [(1, 128)] * 3 + [(128, 128), (1, 128)]],
        out_specs=[pl.BlockSpec((NPG, g1, 128), lambda i: (0, i, 0)),
                   pl.BlockSpec((1, 3, 128), lambda i: (i, 0, 0))],
        out_shape=[jax.ShapeDtypeStruct((NPG, b, 128), f32),
                   jax.ShapeDtypeStruct((nblk1, 3, 128), f32)],
        compiler_params=pltpu.CompilerParams(dimension_semantics=("parallel",)),
    )(hp1, ps1,
      _row2(gin1['nw']), _row2(gin1['nb']), _row2(gin1['nm']), gin1['w2'], _row2(gin1['b2']),
      _row2(p1w), _row2(p1b), _row2(p1m), gin2['w1'], _row2(gin2['b1']))

    node, edgef = pl.pallas_call(
        functools.partial(_p3_kernel, n_rows=n, g_blk=g3, stat_blk_rows=NPG * g1),
        grid=(nblk3,),
        in_specs=[pl.BlockSpec((NPG, g3, 128), lambda i: (0, i, 0)),
                  _full_spec((nblk1, 3, 128))] + [
            _full_spec(s) for s in
            [(1, 128)] * 3 + [(128, 128), (1, 128)]
            + [(128, 128), (1, 128), (128, 128), (1, 128), (128, 4), (1, 4)]
            + [(128, 128), (1, 128), (128, 128), (1, 128), (128, 5), (1, 5)]],
        out_specs=[pl.BlockSpec((NPG, g3, 4), lambda i: (0, i, 0)),
                   pl.BlockSpec((NPG, NPG, g3, 5), lambda i: (0, 0, i, 0))],
        out_shape=[jax.ShapeDtypeStruct((NPG, b, 4), f32),
                   jax.ShapeDtypeStruct((NPG, NPG, b, 5), f32)],
        compiler_params=pltpu.CompilerParams(dimension_semantics=("parallel",)),
    )(hp2, ps2,
      _row2(gin2['nw']), _row2(gin2['nb']), _row2(gin2['nm']), gin2['w2'], _row2(gin2['b2']),
      fw1, _row2(fb1), fw2, _row2(fb2), fw3, _row2(fb3),
      ew1, _row2(eb1), ew2, _row2(eb2), ew3, _row2(eb3))

    node_out = node.transpose(1, 0, 2).reshape(n, 4)
    ef = edgef.transpose(2, 0, 1, 3).reshape(b, NPG * NPG, 5)
    off = np.array([i * NPG + j for i in range(NPG) for j in range(NPG) if i != j])
    dia = np.array([i * (NPG + 1) for i in range(NPG)])
    edge_attr = jnp.concatenate([ef[:, off].reshape(b * 72, 5),
                                 ef[:, dia].reshape(n, 5)], axis=0)
    pred_num = pred.reshape(-1)
    return node_out, edge_attr, pred_num


# speedup vs baseline: 32.1865x; 1.0399x over previous
import functools

import jax
import jax.numpy as jnp
import numpy as np
from jax.experimental import pallas as pl
from jax.experimental.pallas import tpu as pltpu

NPG = 9
EPS = 1e-5
SLOPE = 0.01


def _lap_eig():
    A = np.ones((NPG, NPG)) - np.eye(NPG)
    L = np.diag(A.sum(axis=1)) - A
    _, v = np.linalg.eig(L)
    v = np.real(v)
    v = v / np.linalg.norm(v, axis=0)
    return jnp.asarray(v, jnp.float32)


def _leaky(x):
    return jnp.where(x >= 0, x, SLOPE * x)


def _full_spec(shape):
    nd = len(shape)
    return pl.BlockSpec(shape, lambda i: (0,) * nd)


def _stats(ps_ref, n_rows, blk_rows, nm):
    ps = ps_ref[...]
    s, sd_, sd2 = ps[:, 0], ps[:, 1], ps[:, 2]
    mean = (blk_rows * jnp.sum(s, axis=0, keepdims=True)
            + jnp.sum(sd_, axis=0, keepdims=True)) / n_rows
    delta = mean * nm - s
    var = (jnp.sum(sd2, axis=0, keepdims=True)
           - 2.0 * jnp.sum(delta * sd_, axis=0, keepdims=True)
           + blk_rows * jnp.sum(delta * delta, axis=0, keepdims=True)) / n_rows
    return mean, jnp.sqrt(var + EPS)


def _shifted_partials(hp):
    s = jnp.mean(hp, axis=0, keepdims=True)
    d = hp - s
    return jnp.stack([s[0], jnp.sum(d, axis=0), jnp.sum(d * d, axis=0)])[None]


def _p0_kernel(gv_ref, nw1, nb1, nw2, nb2, nw3, nb3,
               gw1, gb1, gw2, gb2, gw3, gb3, w1a,
               pred_ref, u_ref, ps_ref):
    x = gv_ref[...]
    a = jnp.maximum(jnp.dot(x, nw1[...], preferred_element_type=jnp.float32) + nb1[...], 0.0)
    a = jnp.maximum(jnp.dot(a, nw2[...], preferred_element_type=jnp.float32) + nb2[...], 0.0)
    pred_ref[...] = jnp.dot(a, nw3[...], preferred_element_type=jnp.float32) + nb3[...]
    g = _leaky(jnp.dot(x, gw1[...], preferred_element_type=jnp.float32) + gb1[...])
    g = _leaky(jnp.dot(g, gw2[...], preferred_element_type=jnp.float32) + gb2[...])
    g = jnp.dot(g, gw3[...], preferred_element_type=jnp.float32) + gb3[...]
    agg = g + g
    for _ in range(7):
        agg = agg + g
    u = jnp.dot(g + agg, w1a[...], preferred_element_type=jnp.float32)
    u_ref[...] = u
    ps_ref[...] = jnp.stack([jnp.sum(u, axis=0), jnp.sum(u * u, axis=0)])[None]


def _gin_tail(t, w2, b2, pw, pb, pm, w1n, b1n, g_blk):
    h = jnp.dot(t.reshape(NPG * g_blk, 128), w2[...],
                preferred_element_type=jnp.float32) + b2[...]
    h3 = h.reshape(NPG, g_blk, 128)
    mg = jnp.mean(h3, axis=0, keepdims=True)
    out = h3 - mg * pm[...][None]
    vg = jnp.mean(out * out, axis=0, keepdims=True)
    y = _leaky(pw[...][None] * out / jnp.sqrt(vg + EPS) + pb[...][None])
    hin = y + jnp.sum(y, axis=0, keepdims=True)
    return jnp.dot(hin.reshape(NPG * g_blk, 128), w1n[...],
                   preferred_element_type=jnp.float32) + b1n[...]


def _p1_kernel(u_ref, c_ref, ps0_ref,
               n0w, n0b, n0m, w2_0, b2_0, pw, pb, pm, w1_1, b1_1,
               hp1_ref, ps_ref, *, batch_size, g_blk):
    nm = n0m[...]
    c = c_ref[...]
    sums = jnp.sum(ps0_ref[...], axis=0)
    su, su2 = sums[0:1], sums[1:2]
    sc = jnp.sum(c, axis=0, keepdims=True)
    sc2 = jnp.sum(c * c, axis=0, keepdims=True)
    n_rows = batch_size * NPG
    mean = (NPG * su + batch_size * sc) / n_rows
    eh2 = (NPG * su2 + 2.0 * su * sc + batch_size * sc2) / n_rows
    var = eh2 - mean * mean * nm * (2.0 - nm)
    sd = jnp.sqrt(var + EPS)

    hpre0 = u_ref[...][None, :, :] + c[:, None, :]
    out0 = hpre0 - (mean * nm)[None]
    t = _leaky(n0w[...][None] * out0 / sd[None] + n0b[...][None])
    hp1 = _gin_tail(t, w2_0, b2_0, pw, pb, pm, w1_1, b1_1, g_blk)
    hp1_ref[...] = hp1.reshape(NPG, g_blk, 128)
    ps_ref[...] = _shifted_partials(hp1)


def _p2_kernel(hp_ref, ps_in_ref,
               nw, nb, nm_, w2, b2, pw, pb, pm, w1n, b1n,
               hp_out_ref, ps_ref, *, n_rows, g_blk):
    nm = nm_[...]
    mean, sd = _stats(ps_in_ref, n_rows, NPG * g_blk, nm)
    out0 = hp_ref[...] - (mean * nm)[None]
    t = _leaky(nw[...][None] * out0 / sd[None] + nb[...][None])
    hp = _gin_tail(t, w2, b2, pw, pb, pm, w1n, b1n, g_blk)
    hp_out_ref[...] = hp.reshape(NPG, g_blk, 128)
    ps_ref[...] = _shifted_partials(hp)


def _p3_kernel(hp_ref, ps_in_ref,
               nw, nb, nm_, w2, b2,
               fw1, fb1, fw2, fb2, fw3, fb3,
               ew1, eb1, ew2, eb2, ew3, eb3,
               node_ref, edgef_ref, *, n_rows, g_blk, stat_blk_rows):
    nm = nm_[...]
    mean, sd = _stats(ps_in_ref, n_rows, stat_blk_rows, nm)
    out0 = hp_ref[...] - (mean * nm)[None]
    t = _leaky(nw[...][None] * out0 / sd[None] + nb[...][None])
    h = jnp.dot(t.reshape(NPG * g_blk, 128), w2[...],
                preferred_element_type=jnp.float32) + b2[...]
    a = _leaky(jnp.dot(h, fw1[...], preferred_element_type=jnp.float32) + fb1[...])
    a = _leaky(jnp.dot(a, fw2[...], preferred_element_type=jnp.float32) + fb2[...])
    node = jnp.dot(a, fw3[...], preferred_element_type=jnp.float32) + fb3[...]
    node_ref[...] = node.reshape(NPG, g_blk, 4)
    h3 = h.reshape(NPG, g_blk, 128)
    for i in range(NPG):
        nj = NPG - i
        pin = ((h3[i:] + h3[i:i + 1]) / 2.0).reshape(nj * g_blk, 128)
        e = _leaky(jnp.dot(pin, ew1[...], preferred_element_type=jnp.float32) + eb1[...])
        e = _leaky(jnp.dot(e, ew2[...], preferred_element_type=jnp.float32) + eb2[...])
        e5 = jnp.dot(e, ew3[...], preferred_element_type=jnp.float32) + eb3[...]
        edgef_ref[i, i:] = e5.reshape(nj, g_blk, 5)


def _row2(v):
    return v.reshape(1, -1)


def kernel(global_vec, batch, params):
    del batch
    b = global_vec.shape[0]
    n = b * NPG
    f32 = jnp.float32

    gin0, gin1, gin2 = params['gin']
    (nw1, nb1), (nw2, nb2), (nw3, nb3) = params['num_net']
    (gw1, gb1), (gw2, gb2), (gw3, gb3) = params['glob']
    (p0w, p0b, p0m), (p1w, p1b, p1m) = params['norms']
    (fw1, fb1), (fw2, fb2), (fw3, fb3) = params['feat']
    (ew1, eb1), (ew2, eb2), (ew3, eb3) = params['edge']

    eig = _lap_eig()
    w1a = gin0['w1'][:64]
    c = jnp.dot(eig + jnp.sum(eig, axis=0, keepdims=True),
                gin0['w1'][64:]) + gin0['b1']

    g0 = min(2048, b)
    g1 = min(512, b)
    g3 = min(128, b)
    nblk0 = b // g0
    nblk1 = b // g1
    nblk3 = b // g3

    pred, u, ps0 = pl.pallas_call(
        _p0_kernel,
        grid=(nblk0,),
        in_specs=[pl.BlockSpec((g0, 128), lambda i: (i, 0))] + [
            _full_spec(s) for s in [(128, 128), (1, 128), (128, 128), (1, 128),
                                    (128, 1), (1, 1),
                                    (128, 64), (1, 64), (64, 64), (1, 64),
                                    (64, 64), (1, 64), (64, 128)]],
        out_specs=[pl.BlockSpec((g0, 1), lambda i: (i, 0)),
                   pl.BlockSpec((g0, 128), lambda i: (i, 0)),
                   pl.BlockSpec((1, 2, 128), lambda i: (i, 0, 0))],
        out_shape=[jax.ShapeDtypeStruct((b, 1), f32),
                   jax.ShapeDtypeStruct((b, 128), f32),
                   jax.ShapeDtypeStruct((nblk0, 2, 128), f32)],
        compiler_params=pltpu.CompilerParams(dimension_semantics=("parallel",)),
    )(global_vec, nw1, _row2(nb1), nw2, _row2(nb2), nw3, _row2(nb3),
      gw1, _row2(gb1), gw2, _row2(gb2), gw3, _row2(gb3), w1a)

    hp1, ps1 = pl.pallas_call(
        functools.partial(_p1_kernel, batch_size=b, g_blk=g1),
        grid=(nblk1,),
        in_specs=[pl.BlockSpec((g1, 128), lambda i: (i, 0)),
                  _full_spec((NPG, 128)),
                  _full_spec((nblk0, 2, 128))] + [
            _full_spec(s) for s in [(1, 128)] * 3 + [(128, 128), (1, 128)]
                                    + [(1, 128)] * 3 + [(128, 128), (1, 128)]],
        out_specs=[pl.BlockSpec((NPG, g1, 128), lambda i: (0, i, 0)),
                   pl.BlockSpec((1, 3, 128), lambda i: (i, 0, 0))],
        out_shape=[jax.ShapeDtypeStruct((NPG, b, 128), f32),
                   jax.ShapeDtypeStruct((nblk1, 3, 128), f32)],
        compiler_params=pltpu.CompilerParams(dimension_semantics=("parallel",)),
    )(u, c, ps0,
      _row2(gin0['nw']), _row2(gin0['nb']), _row2(gin0['nm']), gin0['w2'], _row2(gin0['b2']),
      _row2(p0w), _row2(p0b), _row2(p0m), gin1['w1'], _row2(gin1['b1']))

    hp2, ps2 = pl.pallas_call(
        functools.partial(_p2_kernel, n_rows=n, g_blk=g1),
        grid=(nblk1,),
        in_specs=[pl.BlockSpec((NPG, g1, 128), lambda i: (0, i, 0)),
                  _full_spec((nblk1, 3, 128))] + [
            _full_spec(s) for s in [(1, 128)] * 3 + [(128, 128), (1, 128)]
                                    + [(1, 128)] * 3 + [(128, 128), (1, 128)]],
        out_specs=[pl.BlockSpec((NPG, g1, 128), lambda i: (0, i, 0)),
                   pl.BlockSpec((1, 3, 128), lambda i: (i, 0, 0))],
        out_shape=[jax.ShapeDtypeStruct((NPG, b, 128), f32),
                   jax.ShapeDtypeStruct((nblk1, 3, 128), f32)],
        compiler_params=pltpu.CompilerParams(dimension_semantics=("parallel",)),
    )(hp1, ps1,
      _row2(gin1['nw']), _row2(gin1['nb']), _row2(gin1['nm']), gin1['w2'], _row2(gin1['b2']),
      _row2(p1w), _row2(p1b), _row2(p1m), gin2['w1'], _row2(gin2['b1']))

    node, edgef = pl.pallas_call(
        functools.partial(_p3_kernel, n_rows=n, g_blk=g3, stat_blk_rows=NPG * g1),
        grid=(nblk3,),
        in_specs=[pl.BlockSpec((NPG, g3, 128), lambda i: (0, i, 0)),
                  _full_spec((nblk1, 3, 128))] + [
            _full_spec(s) for s in
            [(1, 128)] * 3 + [(128, 128), (1, 128)]
            + [(128, 128), (1, 128), (128, 128), (1, 128), (128, 4), (1, 4)]
            + [(128, 128), (1, 128), (128, 128), (1, 128), (128, 5), (1, 5)]],
        out_specs=[pl.BlockSpec((NPG, g3, 4), lambda i: (0, i, 0)),
                   pl.BlockSpec((NPG, NPG, g3, 5), lambda i: (0, 0, i, 0))],
        out_shape=[jax.ShapeDtypeStruct((NPG, b, 4), f32),
                   jax.ShapeDtypeStruct((NPG, NPG, b, 5), f32)],
        compiler_params=pltpu.CompilerParams(dimension_semantics=("parallel",)),
    )(hp2, ps2,
      _row2(gin2['nw']), _row2(gin2['nb']), _row2(gin2['nm']), gin2['w2'], _row2(gin2['b2']),
      fw1, _row2(fb1), fw2, _row2(fb2), fw3, _row2(fb3),
      ew1, _row2(eb1), ew2, _row2(eb2), ew3, _row2(eb3))

    node_out = node.transpose(1, 0, 2).reshape(n, 4)
    ef2 = edgef.reshape(NPG * NPG * b, 5)
    gs = np.arange(b, dtype=np.int64)
    pieces = [((min(i, j) * NPG + max(i, j)) * b + gs)
              for i in range(NPG) for j in range(NPG) if i != j]
    pieces.append(None)
    cross_idx = np.stack(pieces[:-1], axis=1).reshape(-1)
    loop_idx = np.stack([(i * NPG + i) * b + gs for i in range(NPG)],
                        axis=1).reshape(-1)
    idx = jnp.asarray(np.concatenate([cross_idx, loop_idx]), jnp.int32)
    edge_attr = jnp.take(ef2, idx, axis=0)
    pred_num = pred.reshape(-1)
    return node_out, edge_attr, pred_num
